# Initial kernel scaffold; baseline (speedup 1.0000x reference)
import jax
import jax.numpy as jnp
from jax.experimental import pallas as pl


def kernel(entity_emb, relation_emb, sel_w, sel_b, lin_w, lin_b, h, r, t, edge_index):
    raise NotImplementedError("write your pallas kernel here")


# speedup vs baseline: 2.9916x; 2.9916x over previous
import functools

import jax
import jax.numpy as jnp
from jax import lax
from jax.experimental import pallas as pl
from jax.experimental.pallas import tpu as pltpu
from jax.experimental.pallas import tpu_sc as plsc

NUM_ENT = 50000
NUM_REL = 500
REL_PAD = 512
D = 64
K = 5
B = 16384
NC = 2
NS = 16
NW = NC * NS
CHUNK = B // NW
JCH = 128
NJ = CHUNK // JCH


def _mask_body(rel_ref, w_ref, b_ref, out_ref):
    rel = rel_ref[...]
    w = w_ref[...]
    bias = b_ref[0, 0]
    s = jnp.sum(rel * w, axis=1, keepdims=True) + bias
    rid = lax.broadcasted_iota(jnp.int32, (REL_PAD, 1), 0)
    neg = jnp.float32(-jnp.inf)
    s = jnp.where(rid < NUM_REL, s, neg)
    mask = jnp.zeros((REL_PAD, 1), jnp.float32)
    rem = s
    for _ in range(K):
        cur = jnp.max(rem)
        ismax = rem == cur
        first = jnp.min(jnp.where(ismax, rid, jnp.int32(1 << 30)))
        sel = rid == first
        mask = jnp.where(sel, jnp.float32(1.0), mask)
        rem = jnp.where(sel, neg, rem)
    out_ref[...] = rel * mask


_mask_call = pl.pallas_call(
    _mask_body,
    out_shape=jax.ShapeDtypeStruct((REL_PAD, D), jnp.float32),
)


def _sc_body(ent_hbm, relm_hbm, h_hbm, r_hbm, t_hbm, out_hbm,
             idx_h, idx_r, idx_t, hv, rv, tv, outv,
             s0, s1, s2, s3):
    wid = lax.axis_index("s") * NC + lax.axis_index("c")
    base = wid * CHUNK

    pltpu.sync_copy(h_hbm.at[pl.ds(wid * NJ, NJ)], idx_h)
    pltpu.sync_copy(r_hbm.at[pl.ds(wid * NJ, NJ)], idx_r)
    pltpu.sync_copy(t_hbm.at[pl.ds(wid * NJ, NJ)], idx_t)

    sems = [s0, s1, s2, s3]
    cps = []
    for j in range(NJ):
        dst = pl.ds(j * JCH, JCH)
        cps.append((
            pltpu.async_copy(ent_hbm.at[idx_h.at[j]], hv.at[dst], sems[j]),
            pltpu.async_copy(relm_hbm.at[idx_r.at[j]], rv.at[dst], sems[j]),
            pltpu.async_copy(ent_hbm.at[idx_t.at[j]], tv.at[dst], sems[j]),
        ))

    lanes = lax.iota(jnp.int32, 16)

    def grp_body(g, carry):
        vec = jnp.zeros((16,), jnp.float32)
        for l in range(16):
            i = g * 16 + l
            acc = jnp.zeros((16,), jnp.float32)
            for c in range(D // 16):
                sl = pl.ds(c * 16, 16)
                acc = acc + (hv[i, sl] + rv[i, sl]) * tv[i, sl]
            vec = vec + jnp.where(lanes == l, jnp.sum(acc), jnp.float32(0.0))
        outv[pl.ds(g * 16, 16)] = vec
        return carry

    gpj = JCH // 16
    for j in range(NJ):
        for cp in cps[j]:
            cp.wait()
        lax.fori_loop(j * gpj, (j + 1) * gpj, grp_body, 0)

    pltpu.sync_copy(outv, out_hbm.at[pl.ds(base, CHUNK)])


_sc_call = functools.partial(
    pl.kernel,
    mesh=plsc.VectorSubcoreMesh(core_axis_name="c", subcore_axis_name="s"),
    out_type=jax.ShapeDtypeStruct((B,), jnp.float32),
    compiler_params=pltpu.CompilerParams(
        needs_layout_passes=False, use_tc_tiling_on_sc=False),
    scratch_types=[
        pltpu.VMEM((NJ, JCH), jnp.int32),
        pltpu.VMEM((NJ, JCH), jnp.int32),
        pltpu.VMEM((NJ, JCH), jnp.int32),
        pltpu.VMEM((CHUNK, D), jnp.float32),
        pltpu.VMEM((CHUNK, D), jnp.float32),
        pltpu.VMEM((CHUNK, D), jnp.float32),
        pltpu.VMEM((CHUNK,), jnp.float32),
        pltpu.SemaphoreType.DMA,
        pltpu.SemaphoreType.DMA,
        pltpu.SemaphoreType.DMA,
        pltpu.SemaphoreType.DMA,
    ],
)(_sc_body)


def kernel(entity_emb, relation_emb, sel_w, sel_b, lin_w, lin_b, h, r, t, edge_index):
    relp = jnp.pad(relation_emb, ((0, REL_PAD - NUM_REL), (0, 0)))
    relm = _mask_call(relp, sel_w.reshape(1, D), sel_b.reshape(1, 1))
    h2 = h.reshape(NW * NJ, JCH)
    r2 = r.reshape(NW * NJ, JCH)
    t2 = t.reshape(NW * NJ, JCH)
    return _sc_call(entity_emb, relm, h2, r2, t2)


# speedup vs baseline: 3.2433x; 1.0841x over previous
import functools

import jax
import jax.numpy as jnp
from jax import lax
from jax.experimental import pallas as pl
from jax.experimental.pallas import tpu as pltpu
from jax.experimental.pallas import tpu_sc as plsc

NUM_ENT = 50000
NUM_REL = 500
REL_PAD = 512
D = 64
DP = 128
K = 5
B = 16384
NC = 2
NS = 16
NW = NC * NS
CHUNK = B // NW
JCH = 128
NJ = CHUNK // JCH


def _mask_body(rel_ref, w_ref, b_ref, out_ref):
    rel = rel_ref[...]
    w = w_ref[...]
    bias = b_ref[0, 0]
    s = jnp.sum(rel * w, axis=1, keepdims=True) + bias
    rid = lax.broadcasted_iota(jnp.int32, (REL_PAD, 1), 0)
    neg = jnp.float32(-jnp.inf)
    s = jnp.where(rid < NUM_REL, s, neg)
    mask = jnp.zeros((REL_PAD, 1), jnp.float32)
    rem = s
    for _ in range(K):
        cur = jnp.max(rem)
        ismax = rem == cur
        first = jnp.min(jnp.where(ismax, rid, jnp.int32(1 << 30)))
        sel = rid == first
        mask = jnp.where(sel, jnp.float32(1.0), mask)
        rem = jnp.where(sel, neg, rem)
    out_ref[...] = rel * mask


_mask_call = pl.pallas_call(
    _mask_body,
    out_shape=jax.ShapeDtypeStruct((REL_PAD, D), jnp.float32),
)


def _sc_body(ent_hbm, relm_hbm, h_hbm, r_hbm, t_hbm, out_hbm,
             idx_h, idx_r, idx_t,
             hv0, rv0, tv0, hv1, rv1, tv1, outv, s0, s1):
    wid = lax.axis_index("s") * NC + lax.axis_index("c")
    base = wid * CHUNK

    pltpu.sync_copy(h_hbm.at[pl.ds(base, CHUNK)], idx_h)
    pltpu.sync_copy(r_hbm.at[pl.ds(base, CHUNK)], idx_r)
    pltpu.sync_copy(t_hbm.at[pl.ds(base, CHUNK)], idx_t)

    bufs = ((hv0, rv0, tv0, s0), (hv1, rv1, tv1, s1))
    cps = {}

    def issue(j):
        hb, rb, tb, sem = bufs[j % 2]
        sl = pl.ds(j * JCH, JCH)
        cps[j] = (
            pltpu.async_copy(ent_hbm.at[idx_h.at[sl]], hb, sem),
            pltpu.async_copy(relm_hbm.at[idx_r.at[sl]], rb, sem),
            pltpu.async_copy(ent_hbm.at[idx_t.at[sl]], tb, sem),
        )

    issue(0)
    issue(1)

    lanes = lax.iota(jnp.int32, 16)
    perms = [lanes ^ k for k in (8, 4, 2, 1)]

    for j in range(NJ):
        for cp in cps[j]:
            cp.wait()
        hb, rb, tb, _ = bufs[j % 2]

        def grp_body(g, carry, hb=hb, rb=rb, tb=tb, j=j):
            vec = jnp.zeros((16,), jnp.float32)
            for l in range(16):
                i = g * 16 + l - j * JCH
                acc = jnp.zeros((16,), jnp.float32)
                for c in range(D // 16):
                    sl = pl.ds(c * 16, 16)
                    acc = acc + (hb[i, sl] + rb[i, sl]) * tb[i, sl]
                for p in perms:
                    acc = acc + acc.at[p].get(mode="promise_in_bounds")
                vec = jnp.where(lanes == l, acc, vec)
            outv[pl.ds(g * 16, 16)] = vec
            return carry

        lax.fori_loop(j * (JCH // 16), (j + 1) * (JCH // 16), grp_body, 0)
        if j + 2 < NJ:
            issue(j + 2)

    pltpu.sync_copy(outv, out_hbm.at[pl.ds(base, CHUNK)])


_sc_call = functools.partial(
    pl.kernel,
    mesh=plsc.VectorSubcoreMesh(core_axis_name="c", subcore_axis_name="s"),
    out_type=jax.ShapeDtypeStruct((B,), jnp.float32),
    compiler_params=pltpu.CompilerParams(needs_layout_passes=False),
    scratch_types=[
        pltpu.VMEM((CHUNK,), jnp.int32),
        pltpu.VMEM((CHUNK,), jnp.int32),
        pltpu.VMEM((CHUNK,), jnp.int32),
        pltpu.VMEM((JCH, DP), jnp.float32),
        pltpu.VMEM((JCH, DP), jnp.float32),
        pltpu.VMEM((JCH, DP), jnp.float32),
        pltpu.VMEM((JCH, DP), jnp.float32),
        pltpu.VMEM((JCH, DP), jnp.float32),
        pltpu.VMEM((JCH, DP), jnp.float32),
        pltpu.VMEM((CHUNK,), jnp.float32),
        pltpu.SemaphoreType.DMA,
        pltpu.SemaphoreType.DMA,
    ],
)(_sc_body)


def kernel(entity_emb, relation_emb, sel_w, sel_b, lin_w, lin_b, h, r, t, edge_index):
    relp = jnp.pad(relation_emb, ((0, REL_PAD - NUM_REL), (0, 0)))
    relm = _mask_call(relp, sel_w.reshape(1, D), sel_b.reshape(1, 1))
    relmp = jnp.pad(relm, ((0, 0), (0, DP - D)))
    entp = jnp.pad(entity_emb, ((0, 0), (0, DP - D)))
    return _sc_call(entp, relmp, h, r, t)


# speedup vs baseline: 3.3111x; 1.0209x over previous
import functools

import jax
import jax.numpy as jnp
from jax import lax
from jax.experimental import pallas as pl
from jax.experimental.pallas import tpu as pltpu
from jax.experimental.pallas import tpu_sc as plsc

NUM_ENT = 50000
NUM_REL = 500
REL_PAD = 512
D = 64
DP = 128
K = 5
B = 16384
NC = 2
NS = 16
NW = NC * NS
CHUNK = B // NW
JCH = 128
NJ = CHUNK // JCH


def _mask_body(rel_ref, w_ref, b_ref, out_ref):
    rel = rel_ref[...]
    w = w_ref[...]
    bias = b_ref[0, 0]
    s = jnp.sum(rel * w, axis=1, keepdims=True) + bias
    rid = lax.broadcasted_iota(jnp.int32, (REL_PAD, 1), 0)
    neg = jnp.float32(-jnp.inf)
    s = jnp.where(rid < NUM_REL, s, neg)
    mask = jnp.zeros((REL_PAD, 1), jnp.float32)
    rem = s
    for _ in range(K):
        cur = jnp.max(rem)
        ismax = rem == cur
        first = jnp.min(jnp.where(ismax, rid, jnp.int32(1 << 30)))
        sel = rid == first
        mask = jnp.where(sel, jnp.float32(1.0), mask)
        rem = jnp.where(sel, neg, rem)
    out_ref[...] = rel * mask


_mask_call = pl.pallas_call(
    _mask_body,
    out_shape=jax.ShapeDtypeStruct((REL_PAD, D), jnp.float32),
)


def _sc_body(ent_hbm, relm_hbm, h_hbm, r_hbm, t_hbm, out_hbm,
             idx_h, idx_r, idx_t,
             hv0, rv0, tv0, hv1, rv1, tv1, outv, s0, s1):
    wid = lax.axis_index("s") * NC + lax.axis_index("c")
    base = wid * CHUNK

    pltpu.sync_copy(h_hbm.at[pl.ds(base, CHUNK)], idx_h)
    pltpu.sync_copy(r_hbm.at[pl.ds(base, CHUNK)], idx_r)
    pltpu.sync_copy(t_hbm.at[pl.ds(base, CHUNK)], idx_t)

    bufs = ((hv0, rv0, tv0, s0), (hv1, rv1, tv1, s1))
    cps = {}

    def issue(j):
        hb, rb, tb, sem = bufs[j % 2]
        sl = pl.ds(j * JCH, JCH)
        cps[j] = (
            pltpu.async_copy(ent_hbm.at[idx_h.at[sl]], hb, sem),
            pltpu.async_copy(relm_hbm.at[idx_r.at[sl]], rb, sem),
            pltpu.async_copy(ent_hbm.at[idx_t.at[sl]], tb, sem),
        )

    issue(0)
    issue(1)

    lanes = lax.iota(jnp.int32, 16)
    perms = [lanes ^ k for k in (8, 4, 2, 1)]

    for j in range(NJ):
        for cp in cps[j]:
            cp.wait()
        hb, rb, tb, _ = bufs[j % 2]

        def grp_body(g, carry, hb=hb, rb=rb, tb=tb, j=j):
            vec = jnp.zeros((16,), jnp.float32)
            for l in range(16):
                i = g * 16 + l - j * JCH
                acc = jnp.zeros((16,), jnp.float32)
                for c in range(D // 16):
                    sl = pl.ds(c * 16, 16)
                    acc = acc + (hb[i, sl] + rb[i, sl]) * tb[i, sl]
                for p in perms:
                    acc = acc + acc.at[p].get(mode="promise_in_bounds")
                vec = jnp.where(lanes == l, acc, vec)
            outv[pl.ds(g * 16, 16)] = vec
            return carry

        lax.fori_loop(j * (JCH // 16), (j + 1) * (JCH // 16), grp_body, 0)
        if j + 2 < NJ:
            issue(j + 2)

    pltpu.sync_copy(outv, out_hbm.at[pl.ds(base, CHUNK)])


_sc_call = functools.partial(
    pl.kernel,
    mesh=plsc.VectorSubcoreMesh(core_axis_name="c", subcore_axis_name="s"),
    out_type=jax.ShapeDtypeStruct((B,), jnp.float32),
    compiler_params=pltpu.CompilerParams(
        needs_layout_passes=False, use_tc_tiling_on_sc=False),
    scratch_types=[
        pltpu.VMEM((CHUNK,), jnp.int32),
        pltpu.VMEM((CHUNK,), jnp.int32),
        pltpu.VMEM((CHUNK,), jnp.int32),
        pltpu.VMEM((JCH, D), jnp.float32),
        pltpu.VMEM((JCH, D), jnp.float32),
        pltpu.VMEM((JCH, D), jnp.float32),
        pltpu.VMEM((JCH, D), jnp.float32),
        pltpu.VMEM((JCH, D), jnp.float32),
        pltpu.VMEM((JCH, D), jnp.float32),
        pltpu.VMEM((CHUNK,), jnp.float32),
        pltpu.SemaphoreType.DMA,
        pltpu.SemaphoreType.DMA,
    ],
)(_sc_body)


def kernel(entity_emb, relation_emb, sel_w, sel_b, lin_w, lin_b, h, r, t, edge_index):
    relp = jnp.pad(relation_emb, ((0, REL_PAD - NUM_REL), (0, 0)))
    relm = _mask_call(relp, sel_w.reshape(1, D), sel_b.reshape(1, 1))
    return _sc_call(entity_emb, relm, h, r, t)
